# Initial kernel scaffold; baseline (speedup 1.0000x reference)
import jax
import jax.numpy as jnp
from jax.experimental import pallas as pl


def kernel(x, layer_selector, W, b):
    raise NotImplementedError("write your pallas kernel here")


# speedup vs baseline: 1.3295x; 1.3295x over previous
import functools

import jax
import jax.numpy as jnp
from jax import lax
from jax.experimental import pallas as pl
from jax.experimental.pallas import tpu as pltpu
from jax.experimental.pallas import tpu_sc as plsc

BATCH = 16384
OUT_N = 64
_LANES = 16


def _make_sc_kernel(batch, out_n):
    info = plsc.get_sparse_core_info()
    nc, ns = info.num_cores, info.num_subcores
    nw = nc * ns
    rows_per_worker = batch // nw
    steps = rows_per_worker // _LANES

    mesh = plsc.VectorSubcoreMesh(core_axis_name="c", subcore_axis_name="s")

    @functools.partial(
        pl.kernel,
        mesh=mesh,
        out_type=jax.ShapeDtypeStruct((batch,), jnp.float32),
        compiler_params=pltpu.CompilerParams(needs_layout_passes=False),
        scratch_types=[
            pltpu.VMEM((rows_per_worker,), jnp.float32),
            pltpu.VMEM((rows_per_worker,), jnp.int32),
            pltpu.VMEM((out_n,), jnp.float32),
            pltpu.VMEM((out_n,), jnp.float32),
            pltpu.VMEM((rows_per_worker,), jnp.float32),
        ],
    )
    def sc_kernel(x_hbm, sel_hbm, w_hbm, b_hbm, out_hbm,
                  x_v, sel_v, w_v, b_v, out_v):
        wid = lax.axis_index("s") * nc + lax.axis_index("c")
        base = wid * rows_per_worker
        pltpu.sync_copy(x_hbm.at[pl.ds(base, rows_per_worker)], x_v)
        pltpu.sync_copy(sel_hbm.at[pl.ds(base, rows_per_worker)], sel_v)
        pltpu.sync_copy(w_hbm, w_v)
        pltpu.sync_copy(b_hbm, b_v)

        def body(i, carry):
            off = i * _LANES
            sel = sel_v[pl.ds(off, _LANES)]
            xv = x_v[pl.ds(off, _LANES)]
            idx = jnp.maximum(sel, 0)
            wv = plsc.load_gather(w_v, [idx])
            bv = plsc.load_gather(b_v, [idx])
            t = xv * wv + bv
            sig = 1.0 / (1.0 + jnp.exp(-t))
            out_v[pl.ds(off, _LANES)] = jnp.where(sel == -1, xv, sig)
            return carry

        lax.fori_loop(0, steps, body, 0)
        pltpu.sync_copy(out_v, out_hbm.at[pl.ds(base, rows_per_worker)])

    return sc_kernel


_SC_KERNEL = None


def kernel(x, layer_selector, W, b):
    global _SC_KERNEL
    if _SC_KERNEL is None:
        _SC_KERNEL = _make_sc_kernel(BATCH, OUT_N)
    xf = x.reshape(-1)
    sel = layer_selector.astype(jnp.int32)
    wf = W.reshape(-1)
    out = _SC_KERNEL(xf, sel, wf, b)
    return out[:, None]
